# baseline (speedup 1.0000x reference)
import jax
import jax.numpy as jnp
import numpy as np
from jax.experimental import pallas as pl
from jax.experimental.pallas import tpu as pltpu

_Z = 1_000_000
_NL = 40
_BLK = 8000
_G = _Z // _BLK
_ABS_SCINT = 39500.0
_ABS_OTHER = 4220000.0
_I0 = np.int32(0)


def _fused_block_kernel(x_ref, bd_ref, par_ref, w_ref, b_ref,
                        prof_ref, part_ref, carry_ref):
    i = pl.program_id(0)
    dz = par_ref[0, 0]
    st = par_ref[0, 1]

    w = w_ref[...]
    x = x_ref[...]
    y = jnp.sum(w * x, axis=1, keepdims=True) + b_ref[...]
    prof = jnp.logaddexp(y, 0.0)
    prof_ref[...] = prof

    ii = jax.lax.broadcasted_iota(jnp.int32, (_BLK, 1), 0) + i * _BLK
    fi = ii.astype(jnp.float32)
    mid = (fi + 0.5) * dz
    midn = (fi + 1.5) * dz
    valid = ii < (_Z - 1)

    bd = bd_ref[...]
    col = jax.lax.broadcasted_iota(jnp.int32, (1, _NL), 1)
    m39 = (col < (_NL - 1)).astype(jnp.float32)
    le = (bd <= mid).astype(jnp.float32)
    cnt40 = jnp.sum(le, axis=1, keepdims=True)
    cnt39 = jnp.sum(le * m39, axis=1, keepdims=True)
    len_ = (bd <= midn).astype(jnp.float32)
    cnt39n = jnp.sum(len_ * m39, axis=1, keepdims=True)
    interface = cnt39n > cnt39
    parity = jnp.mod(cnt40 + st, 2.0)
    is_scint = (parity == 1.0) & jnp.logical_not(interface) & valid

    inv_s = np.float32(1.0 / _ABS_SCINT)
    inv_o = np.float32(1.0 / _ABS_OTHER)
    s_arr = jnp.clip(jnp.ceil(bd / dz - 0.5), 0.0, np.float32(_Z - 1))
    k = (col + 1).astype(jnp.float32)
    kpar = jnp.mod(k + st, 2.0)
    d = 1.0 - 2.0 * kpar
    ip1 = fi + 1.0
    raw = jnp.sum(d * jnp.minimum(ip1, s_arr), axis=1, keepdims=True)
    raw = raw + jnp.mod(st, 2.0) * ip1
    e_mask = (kpar == 0.0) & (col < (_NL - 1))
    corr = jnp.sum(((s_arr - 1.0) <= fi).astype(jnp.float32)
                   * e_mask.astype(jnp.float32), axis=1, keepdims=True)
    ns = raw - corr
    cum = dz * (inv_o * ip1 + (inv_s - inv_o) * ns)
    emis = prof * is_scint.astype(jnp.float32) * jnp.exp(-cum)

    ps = jnp.sum(prof)
    pt = jnp.sum(emis)
    s_acc = jnp.where(i == 0, ps, carry_ref[0, 1] + ps)
    t_acc = jnp.where(i == 0, pt, carry_ref[0, 2] + pt)
    carry_ref[0, 1] = s_acc
    carry_ref[0, 2] = t_acc

    @pl.when(i == _G - 1)
    def _():
        lane = jax.lax.broadcasted_iota(jnp.int32, (1, 128), 1)
        part_ref[...] = jnp.where(lane == 0, s_acc,
                                  jnp.where(lane == 1, t_acc, 0.0))


def kernel(layer_type, batch_size, layer_vec, W, b):
    del batch_size
    st = layer_type[0].astype(jnp.int32)
    xvec = (layer_vec * (1.0 + 0.1 * layer_type.astype(layer_vec.dtype))
            ).astype(jnp.float32).reshape(1, _NL)

    k2 = jnp.arange(_NL // 2)
    swapped = jnp.zeros((_NL,), jnp.float32)
    swapped = swapped.at[st + 2 * k2].set(layer_vec[(1 - st) + 2 * k2])
    swapped = swapped.at[(1 - st) + 2 * k2].set(layer_vec[st + 2 * k2])
    bd = jnp.cumsum(swapped).reshape(1, _NL)

    total = jnp.sum(layer_vec)
    dz = (total / (_Z - 1)).astype(jnp.float32)
    params = jnp.stack([dz, st.astype(jnp.float32)]).reshape(1, 2)

    b2d = b.astype(jnp.float32).reshape(_Z, 1)

    prof, parts = pl.pallas_call(
        _fused_block_kernel,
        grid=(_G,),
        in_specs=[
            pl.BlockSpec((1, _NL), lambda i: (_I0, _I0)),
            pl.BlockSpec((1, _NL), lambda i: (_I0, _I0)),
            pl.BlockSpec((1, 2), lambda i: (_I0, _I0)),
            pl.BlockSpec((_BLK, _NL), lambda i: (i, _I0)),
            pl.BlockSpec((_BLK, 1), lambda i: (i, _I0)),
        ],
        out_specs=[
            pl.BlockSpec((_BLK, 1), lambda i: (i, _I0)),
            pl.BlockSpec((1, 128), lambda i: (_I0, _I0)),
        ],
        out_shape=[
            jax.ShapeDtypeStruct((_Z, 1), jnp.float32),
            jax.ShapeDtypeStruct((1, 128), jnp.float32),
        ],
        scratch_shapes=[pltpu.SMEM((1, 4), jnp.float32)],
        compiler_params=pltpu.CompilerParams(
            dimension_semantics=("arbitrary",)),
    )(xvec, bd, params, W.astype(jnp.float32), b2d)

    profile = prof.reshape(_Z).astype(jnp.float64)
    s = parts[0, 0].astype(jnp.float64)
    tot = parts[0, 1].astype(jnp.float64)

    theta = jnp.linspace(0.0, np.pi / 2.0, 64).astype(jnp.float64)
    phi = jnp.linspace(0.0, 2.0 * np.pi, 64).astype(jnp.float64)
    emission_theta_1 = s * (tot * jnp.cos(theta))
    emission_phi_1 = s * (tot * jnp.ones_like(phi) / (2.0 * np.pi))
    return (profile, theta, emission_theta_1, phi, emission_phi_1)
